# Initial kernel scaffold; baseline (speedup 1.0000x reference)
import jax
import jax.numpy as jnp
from jax.experimental import pallas as pl


def kernel(ent_embed, rel_embed, norm, W_rel_0, W_loop_0, W_rel_1, W_loop_1, edge_index, rel_id):
    raise NotImplementedError("write your pallas kernel here")


# speedup vs baseline: 2.6850x; 2.6850x over previous
import functools

import jax
import jax.numpy as jnp
from jax import lax
from jax.experimental import pallas as pl
from jax.experimental.pallas import tpu as pltpu
from jax.experimental.pallas import tpu_sc as plsc

N = 10000
D = 128
R = 200
E = 320000
SLOPE = (1.0 / 8.0 + 1.0 / 3.0) / 2.0

NPAD = 10240
GPAD = 10400
ROWS_PER_TILE = NPAD // 16
ROW_CHUNK = 128
CH = 128
NW = 32
EP = 2 * E
CPT = (EP + NW * CH - 1) // (NW * CH)
EPAD = NW * CPT * CH


def _dotT(x, w):
    return lax.dot_general(x, w, (((1,), (1,)), ((), ())),
                           preferred_element_type=jnp.float32)


def _layer0_body(h_ref, rel_ref, wr_ref, wl_ref, g_ref, s_ref):
    i = pl.program_id(0)
    x = jnp.where(i < 50, h_ref[...], rel_ref[...])
    g_ref[...] = _dotT(x, wr_ref[...])
    s_ref[...] = _dotT(x, wl_ref[...])


def _layer0_tc(ent_embed, rel_embed, wr, wl):
    return pl.pallas_call(
        _layer0_body,
        grid=(51,),
        in_specs=[
            pl.BlockSpec((200, D), lambda i: (jnp.minimum(i, 49), 0)),
            pl.BlockSpec((200, D), lambda i: (0, 0)),
            pl.BlockSpec((D, D), lambda i: (0, 0)),
            pl.BlockSpec((D, D), lambda i: (0, 0)),
        ],
        out_specs=[
            pl.BlockSpec((200, D), lambda i: (i, 0)),
            pl.BlockSpec((200, D), lambda i: (i, 0)),
        ],
        out_shape=[
            jax.ShapeDtypeStruct((GPAD, D), jnp.float32),
            jax.ShapeDtypeStruct((GPAD, D), jnp.float32),
        ],
    )(ent_embed, rel_embed, wr, wl)


def _layer1_body(pa_ref, pb_ref, norm_ref, s0_ref, rel_ref, wr_ref, wl_ref,
                 g_ref, s1_ref):
    i = pl.program_id(0)
    h = (pa_ref[0] + pb_ref[0]) * norm_ref[...] + s0_ref[...]
    h = jnp.where(h >= 0, h, SLOPE * h)
    relx = jnp.concatenate([rel_ref[...], rel_ref[...]], axis=0)
    x = jnp.where(i < 25, h, relx)
    g_ref[...] = _dotT(x, wr_ref[...])
    s1_ref[...] = _dotT(x, wl_ref[...])


def _layer1_tc(p, norm, s0, rel_embed, wr, wl):
    return pl.pallas_call(
        _layer1_body,
        grid=(26,),
        in_specs=[
            pl.BlockSpec((1, 400, D), lambda i: (0, jnp.minimum(i, 24), 0)),
            pl.BlockSpec((1, 400, D), lambda i: (1, jnp.minimum(i, 24), 0)),
            pl.BlockSpec((400, 1), lambda i: (jnp.minimum(i, 24), 0)),
            pl.BlockSpec((400, D), lambda i: (jnp.minimum(i, 24), 0)),
            pl.BlockSpec((200, D), lambda i: (0, 0)),
            pl.BlockSpec((D, D), lambda i: (0, 0)),
            pl.BlockSpec((D, D), lambda i: (0, 0)),
        ],
        out_specs=[
            pl.BlockSpec((400, D), lambda i: (i, 0)),
            pl.BlockSpec((400, D), lambda i: (i, 0)),
        ],
        out_shape=[
            jax.ShapeDtypeStruct((GPAD, D), jnp.float32),
            jax.ShapeDtypeStruct((GPAD, D), jnp.float32),
        ],
    )(p, p, norm, s0, rel_embed, wr, wl)


def _final_body(pa_ref, pb_ref, norm_ref, s1_ref, o_ref):
    h = (pa_ref[0] + pb_ref[0]) * norm_ref[...] + s1_ref[...]
    o_ref[...] = jnp.where(h >= 0, h, SLOPE * h)


def _final_tc(p, norm, s1):
    return pl.pallas_call(
        _final_body,
        grid=(25,),
        in_specs=[
            pl.BlockSpec((1, 400, D), lambda i: (0, i, 0)),
            pl.BlockSpec((1, 400, D), lambda i: (1, i, 0)),
            pl.BlockSpec((400, 1), lambda i: (i, 0)),
            pl.BlockSpec((400, D), lambda i: (i, 0)),
        ],
        out_specs=pl.BlockSpec((400, D), lambda i: (i, 0)),
        out_shape=jax.ShapeDtypeStruct((N, D), jnp.float32),
    )(p, p, norm, s1)


_SC_MESH = plsc.VectorSubcoreMesh(core_axis_name="c", subcore_axis_name="s")


@functools.partial(
    pl.kernel,
    mesh=_SC_MESH,
    out_type=jax.ShapeDtypeStruct((2, NPAD, D), jnp.float32),
    scratch_types=[
        pltpu.VMEM((CH,), jnp.int32),
        pltpu.VMEM((CH,), jnp.int32),
        pltpu.VMEM((CH, D), jnp.float32),
        pltpu.VMEM((ROW_CHUNK, D), jnp.float32),
        pltpu.VMEM_SHARED((NPAD, D), jnp.float32),
        pltpu.SemaphoreType.DMA,
    ],
)
def _sc_scatter(g_hbm, gidx_hbm, dst_hbm, out_hbm,
                idx_v, dst_v, rows_v, stage_v, acc_sh, sem):
    cid = lax.axis_index("c")
    sid = lax.axis_index("s")
    wid = cid * 16 + sid

    def zfill(i, carry):
        r = i // 8
        c = (i % 8) * 16
        stage_v[r, pl.ds(c, 16)] = jnp.zeros((16,), jnp.float32)
        return carry
    lax.fori_loop(0, ROW_CHUNK * 8, zfill, 0)
    rowbase = sid * ROWS_PER_TILE

    def zcopy(k, carry):
        pltpu.sync_copy(stage_v, acc_sh.at[pl.ds(rowbase + k * ROW_CHUNK, ROW_CHUNK)])
        return carry
    lax.fori_loop(0, ROWS_PER_TILE // ROW_CHUNK, zcopy, 0)
    plsc.subcore_barrier()

    ebase = wid * CPT * CH

    def body(i, carry):
        b = ebase + i * CH
        pltpu.sync_copy(gidx_hbm.at[pl.ds(b, CH)], idx_v)
        pltpu.sync_copy(dst_hbm.at[pl.ds(b, CH)], dst_v)
        pltpu.async_copy(g_hbm.at[idx_v], rows_v, sem).wait()
        pltpu.sync_copy(rows_v, acc_sh.at[dst_v], add=True)
        return carry
    lax.fori_loop(0, CPT, body, 0)
    plsc.subcore_barrier()

    def ocopy(k, carry):
        pltpu.sync_copy(acc_sh.at[pl.ds(rowbase + k * ROW_CHUNK, ROW_CHUNK)], stage_v)
        pltpu.sync_copy(stage_v, out_hbm.at[cid, pl.ds(rowbase + k * ROW_CHUNK, ROW_CHUNK)])
        return carry
    lax.fori_loop(0, ROWS_PER_TILE // ROW_CHUNK, ocopy, 0)


def kernel(ent_embed, rel_embed, norm, W_rel_0, W_loop_0, W_rel_1, W_loop_1,
           edge_index, rel_id):
    src = edge_index[0]
    dst = edge_index[1]
    gidx = jnp.stack([src, rel_id + N], axis=1).reshape(-1)
    ddst = jnp.stack([dst, dst], axis=1).reshape(-1)
    gidx = jnp.concatenate([gidx, jnp.zeros((EPAD - EP,), jnp.int32)])
    ddst = jnp.concatenate([ddst, jnp.full((EPAD - EP,), N, jnp.int32)])

    g0, s0 = _layer0_tc(ent_embed, rel_embed, W_rel_0, W_loop_0)
    p0 = _sc_scatter(g0, gidx, ddst)
    g1, s1 = _layer1_tc(p0, norm, s0, rel_embed, W_rel_1, W_loop_1)
    p1 = _sc_scatter(g1, gidx, ddst)
    return _final_tc(p1, norm, s1)
